# initial kernel scaffold (unmeasured)
import jax
import jax.numpy as jnp
from jax import lax
from jax.experimental import pallas as pl
from jax.experimental.pallas import tpu as pltpu

N_DEV = 16
B = 2
S_PER = 128
S = S_PER * N_DEV
D = 512
H_PER = 8
DH = 64
QBLK = 512
SCALE = 0.125


def kernel(x, Wq, Wo, Wk, Wv):
    def body(x_ref, wq_ref, wo_ref, wk_ref, wv_ref, out_ref,
             xfull_ref, q_ref, k_ref, v_ref, attn_ref, part_ref,
             sendbuf_ref, rsrecv_ref,
             ag_send, ag_recv, rs_send, rs_recv):
        me = lax.axis_index("i")
        left = lax.rem(me - 1 + N_DEV, N_DEV)
        right = lax.rem(me + 1, N_DEV)

        barrier = pltpu.get_barrier_semaphore()
        for nbr in (left, right):
            pl.semaphore_signal(barrier, inc=1, device_id=(nbr,),
                                device_id_type=pl.DeviceIdType.MESH)
        pl.semaphore_wait(barrier, 2)

        xfull_ref[:, pl.ds(me * S_PER, S_PER), :] = x_ref[...]
        for h in range(N_DEV - 1):
            org = lax.rem(me - h + 2 * N_DEV, N_DEV)
            rdma = pltpu.make_async_remote_copy(
                src_ref=xfull_ref.at[:, pl.ds(org * S_PER, S_PER), :],
                dst_ref=xfull_ref.at[:, pl.ds(org * S_PER, S_PER), :],
                send_sem=ag_send.at[h],
                recv_sem=ag_recv.at[h],
                device_id=(right,),
                device_id_type=pl.DeviceIdType.MESH,
            )
            rdma.start()
            rdma.wait()

        xf = xfull_ref[...].reshape(B * S, D)
        q_ref[...] = jnp.dot(xf, wq_ref[...], preferred_element_type=jnp.float32)
        k_ref[...] = jnp.dot(xf, wk_ref[...], preferred_element_type=jnp.float32)
        v_ref[...] = jnp.dot(xf, wv_ref[...], preferred_element_type=jnp.float32)

        for b in range(B):
            for hd in range(H_PER):
                c0, c1 = hd * DH, (hd + 1) * DH
                Kb = k_ref[b * S:(b + 1) * S, c0:c1]
                Vb = v_ref[b * S:(b + 1) * S, c0:c1]

                def qstep(qb, _, b=b, c0=c0, c1=c1, Kb=Kb, Vb=Vb):
                    r0 = b * S + qb * QBLK
                    Qb = q_ref[pl.ds(r0, QBLK), c0:c1]
                    s = lax.dot_general(
                        Qb, Kb, (((1,), (1,)), ((), ())),
                        preferred_element_type=jnp.float32) * SCALE
                    m = jnp.max(s, axis=1, keepdims=True)
                    p = jnp.exp(s - m)
                    l = jnp.sum(p, axis=1, keepdims=True)
                    o = lax.dot_general(
                        p, Vb, (((1,), (0,)), ((), ())),
                        preferred_element_type=jnp.float32)
                    attn_ref[pl.ds(r0, QBLK), c0:c1] = o / l
                    return 0

                lax.fori_loop(0, S // QBLK, qstep, 0)

        part = jnp.dot(attn_ref[...], wo_ref[...],
                       preferred_element_type=jnp.float32)
        part_ref[...] = part.reshape(B, S, D)

        for t in range(N_DEV - 1):
            c = lax.rem(me - t - 1 + 2 * N_DEV, N_DEV)
            acc = part_ref[:, pl.ds(c * S_PER, S_PER), :]
            if t > 0:
                acc = acc + rsrecv_ref[t - 1]
            sendbuf_ref[...] = acc
            rdma = pltpu.make_async_remote_copy(
                src_ref=sendbuf_ref,
                dst_ref=rsrecv_ref.at[t],
                send_sem=rs_send.at[t],
                recv_sem=rs_recv.at[t],
                device_id=(right,),
                device_id_type=pl.DeviceIdType.MESH,
            )
            rdma.start()
            rdma.wait()

        out_ref[...] = (part_ref[:, pl.ds(me * S_PER, S_PER), :]
                        + rsrecv_ref[N_DEV - 2])

    return pl.pallas_call(
        body,
        out_shape=jax.ShapeDtypeStruct((B, S_PER, D), jnp.float32),
        in_specs=[pl.BlockSpec(memory_space=pltpu.VMEM)] * 5,
        out_specs=pl.BlockSpec(memory_space=pltpu.VMEM),
        scratch_shapes=[
            pltpu.VMEM((B, S, D), jnp.float32),
            pltpu.VMEM((B * S, D), jnp.float32),
            pltpu.VMEM((B * S, D), jnp.float32),
            pltpu.VMEM((B * S, D), jnp.float32),
            pltpu.VMEM((B * S, D), jnp.float32),
            pltpu.VMEM((B, S, D), jnp.float32),
            pltpu.VMEM((B, S_PER, D), jnp.float32),
            pltpu.VMEM((N_DEV - 1, B, S_PER, D), jnp.float32),
            pltpu.SemaphoreType.DMA((N_DEV - 1,)),
            pltpu.SemaphoreType.DMA((N_DEV - 1,)),
            pltpu.SemaphoreType.DMA((N_DEV - 1,)),
            pltpu.SemaphoreType.DMA((N_DEV - 1,)),
        ],
        compiler_params=pltpu.CompilerParams(collective_id=0),
    )(x, Wq, Wo, Wk, Wv)


# baseline (device time: 450692 ns/iter reference)
import jax
import jax.numpy as jnp
from jax import lax
from jax.experimental import pallas as pl
from jax.experimental.pallas import tpu as pltpu

N_DEV = 16
B = 2
S_PER = 128
S = S_PER * N_DEV
D = 512
H_PER = 8
DH = 64
QBLK = 512
SCALE = 0.125


def kernel(x, Wq, Wo, Wk, Wv):
    def body(x_ref, wq_ref, wo_ref, wk_ref, wv_ref, out_ref,
             xfull_ref, k_ref, v_ref, part_ref,
             sendbuf_ref, rsrecv_ref,
             ag_send, ag_recv, rs_send, rs_recv):
        me = lax.axis_index("i")
        left = lax.rem(me - 1 + N_DEV, N_DEV)
        right = lax.rem(me + 1, N_DEV)

        barrier = pltpu.get_barrier_semaphore()
        for nbr in (left, right):
            pl.semaphore_signal(barrier, inc=1, device_id=(nbr,),
                                device_id_type=pl.DeviceIdType.MESH)
        pl.semaphore_wait(barrier, 2)

        xfull_ref[:, pl.ds(me * S_PER, S_PER), :] = x_ref[...]
        for h in range(N_DEV - 1):
            org = lax.rem(me - h + 2 * N_DEV, N_DEV)
            rdma = pltpu.make_async_remote_copy(
                src_ref=xfull_ref.at[:, pl.ds(org * S_PER, S_PER), :],
                dst_ref=xfull_ref.at[:, pl.ds(org * S_PER, S_PER), :],
                send_sem=ag_send.at[h],
                recv_sem=ag_recv.at[h],
                device_id=(right,),
                device_id_type=pl.DeviceIdType.MESH,
            )
            rdma.start()
            rdma.wait()

        for b in range(B):
            xb = xfull_ref[b]
            k_ref[b] = jnp.dot(xb, wk_ref[...], preferred_element_type=jnp.float32)
            v_ref[b] = jnp.dot(xb, wv_ref[...], preferred_element_type=jnp.float32)

        for b in range(B):
            for hd in range(H_PER):
                c0, c1 = hd * DH, (hd + 1) * DH
                Kb = k_ref[b, :, c0:c1]
                Vb = v_ref[b, :, c0:c1]
                wq_h = wq_ref[:, c0:c1]
                wo_h = wo_ref[c0:c1, :]

                def qstep(qb, _, b=b, hd=hd, Kb=Kb, Vb=Vb, wq_h=wq_h, wo_h=wo_h):
                    r0 = qb * QBLK
                    xq = xfull_ref[b, pl.ds(r0, QBLK), :]
                    Qb = jnp.dot(xq, wq_h,
                                 preferred_element_type=jnp.float32)
                    s = lax.dot_general(
                        Qb, Kb, (((1,), (1,)), ((), ())),
                        preferred_element_type=jnp.float32) * SCALE
                    m = jnp.max(s, axis=1, keepdims=True)
                    p = jnp.exp(s - m)
                    l = jnp.sum(p, axis=1, keepdims=True)
                    o = lax.dot_general(
                        p, Vb, (((1,), (0,)), ((), ())),
                        preferred_element_type=jnp.float32) / l
                    contrib = jnp.dot(o, wo_h,
                                      preferred_element_type=jnp.float32)
                    if hd == 0:
                        part_ref[b, pl.ds(r0, QBLK), :] = contrib
                    else:
                        part_ref[b, pl.ds(r0, QBLK), :] = (
                            part_ref[b, pl.ds(r0, QBLK), :] + contrib)
                    return 0

                lax.fori_loop(0, S // QBLK, qstep, 0)

        for t in range(N_DEV - 1):
            c = lax.rem(me - t - 1 + 2 * N_DEV, N_DEV)
            acc = part_ref[:, pl.ds(c * S_PER, S_PER), :]
            if t > 0:
                acc = acc + rsrecv_ref[t - 1]
            sendbuf_ref[...] = acc
            rdma = pltpu.make_async_remote_copy(
                src_ref=sendbuf_ref,
                dst_ref=rsrecv_ref.at[t],
                send_sem=rs_send.at[t],
                recv_sem=rs_recv.at[t],
                device_id=(right,),
                device_id_type=pl.DeviceIdType.MESH,
            )
            rdma.start()
            rdma.wait()

        out_ref[...] = (part_ref[:, pl.ds(me * S_PER, S_PER), :]
                        + rsrecv_ref[N_DEV - 2])

    return pl.pallas_call(
        body,
        out_shape=jax.ShapeDtypeStruct((B, S_PER, D), jnp.float32),
        in_specs=[pl.BlockSpec(memory_space=pltpu.VMEM)] * 5,
        out_specs=pl.BlockSpec(memory_space=pltpu.VMEM),
        scratch_shapes=[
            pltpu.VMEM((B, S, D), jnp.float32),
            pltpu.VMEM((B, S, D), jnp.float32),
            pltpu.VMEM((B, S, D), jnp.float32),
            pltpu.VMEM((B, S, D), jnp.float32),
            pltpu.VMEM((B, S_PER, D), jnp.float32),
            pltpu.VMEM((N_DEV - 1, B, S_PER, D), jnp.float32),
            pltpu.SemaphoreType.DMA((N_DEV - 1,)),
            pltpu.SemaphoreType.DMA((N_DEV - 1,)),
            pltpu.SemaphoreType.DMA((N_DEV - 1,)),
            pltpu.SemaphoreType.DMA((N_DEV - 1,)),
        ],
        compiler_params=pltpu.CompilerParams(
            collective_id=0, vmem_limit_bytes=100 * 1024 * 1024),
    )(x, Wq, Wo, Wk, Wv)


# device time: 358908 ns/iter; 1.2557x vs baseline; 1.2557x over previous
import jax
import jax.numpy as jnp
from jax import lax
from jax.experimental import pallas as pl
from jax.experimental.pallas import tpu as pltpu

N_DEV = 16
B = 2
S_PER = 128
S = S_PER * N_DEV
D = 512
H_PER = 8
DH = 64
SCALE = 0.125
BF16 = jnp.bfloat16
F32 = jnp.float32


def kernel(x, Wq, Wo, Wk, Wv):
    wq_hm = Wq.reshape(D, H_PER, DH).transpose(1, 0, 2).astype(BF16)
    wk_hm = Wk.reshape(D, H_PER, DH).transpose(1, 0, 2).astype(BF16)
    wv_hm = Wv.reshape(D, H_PER, DH).transpose(1, 0, 2).astype(BF16)
    wo_hm = Wo.reshape(H_PER, DH, D).astype(BF16)

    def body(x_ref, wq_ref, wo_ref, wk_ref, wv_ref, out_ref,
             xfull_ref, k_ref, v_ref, sendbuf_ref, rsrecv_ref,
             ag_send, ag_recv, rs_send, rs_recv):
        me = lax.axis_index("i")
        left = lax.rem(me - 1 + N_DEV, N_DEV)
        right = lax.rem(me + 1, N_DEV)

        barrier = pltpu.get_barrier_semaphore()
        for nbr in (left, right):
            pl.semaphore_signal(barrier, inc=1, device_id=(nbr,),
                                device_id_type=pl.DeviceIdType.MESH)
        pl.semaphore_wait(barrier, 2)

        def kv_chunk(c):
            for b in range(B):
                xc = xfull_ref[b, pl.ds(c * S_PER, S_PER), :]

                def kvb(hd, _):
                    k_ref[b, hd, pl.ds(c * S_PER, S_PER), :] = jnp.dot(
                        xc, wk_ref[hd],
                        preferred_element_type=F32).astype(BF16)
                    v_ref[b, hd, pl.ds(c * S_PER, S_PER), :] = jnp.dot(
                        xc, wv_ref[hd],
                        preferred_element_type=F32).astype(BF16)
                    return 0

                lax.fori_loop(0, H_PER, kvb, 0)

        xfull_ref[:, pl.ds(me * S_PER, S_PER), :] = x_ref[...].astype(BF16)
        ag_rdmas = []
        for h in range(N_DEV - 1):
            org = lax.rem(me - h + 2 * N_DEV, N_DEV)
            rdma = pltpu.make_async_remote_copy(
                src_ref=xfull_ref.at[:, pl.ds(org * S_PER, S_PER), :],
                dst_ref=xfull_ref.at[:, pl.ds(org * S_PER, S_PER), :],
                send_sem=ag_send.at[h],
                recv_sem=ag_recv.at[h],
                device_id=(right,),
                device_id_type=pl.DeviceIdType.MESH,
            )
            if h > 0:
                ag_rdmas[h - 1].wait_recv()
            rdma.start()
            ag_rdmas.append(rdma)
            kv_chunk(org)
        ag_rdmas[-1].wait_recv()
        kv_chunk(lax.rem(me + 1, N_DEV))

        rs_rdmas = []
        for t in range(N_DEV):
            c = lax.rem(me - t - 1 + 2 * N_DEV, N_DEV)
            contribs = []
            for b in range(B):
                xq = xfull_ref[b, pl.ds(c * S_PER, S_PER), :]

                def hd_body(hd, acc, b=b, xq=xq):
                    Qh = jnp.dot(xq, wq_ref[hd],
                                 preferred_element_type=F32).astype(BF16)
                    s = lax.dot_general(
                        Qh, k_ref[b, hd], (((1,), (1,)), ((), ())),
                        preferred_element_type=F32) * SCALE
                    m = jnp.max(s, axis=1, keepdims=True)
                    p = jnp.exp(s - m)
                    l = jnp.sum(p, axis=1, keepdims=True)
                    o = lax.dot_general(
                        p.astype(BF16), v_ref[b, hd],
                        (((1,), (0,)), ((), ())),
                        preferred_element_type=F32) / l
                    return acc + jnp.dot(o.astype(BF16), wo_ref[hd],
                                         preferred_element_type=F32)

                contribs.append(lax.fori_loop(
                    0, H_PER, hd_body, jnp.zeros((S_PER, D), F32)))

            if t < N_DEV - 1:
                slot = t % 2
                if t >= 2:
                    rs_rdmas[t - 2].wait_send()
                if t > 0:
                    rs_rdmas[t - 1].wait_recv()
                for b in range(B):
                    acc = contribs[b]
                    if t > 0:
                        acc = acc + rsrecv_ref[t - 1, b]
                    sendbuf_ref[slot, b] = acc
                rdma = pltpu.make_async_remote_copy(
                    src_ref=sendbuf_ref.at[slot],
                    dst_ref=rsrecv_ref.at[t],
                    send_sem=rs_send.at[t],
                    recv_sem=rs_recv.at[t],
                    device_id=(right,),
                    device_id_type=pl.DeviceIdType.MESH,
                )
                rdma.start()
                rs_rdmas.append(rdma)
            else:
                rs_rdmas[N_DEV - 2].wait_recv()
                for b in range(B):
                    out_ref[b] = contribs[b] + rsrecv_ref[N_DEV - 2, b]

        for r in ag_rdmas:
            r.wait_send()
        rs_rdmas[-2].wait_send()
        rs_rdmas[-1].wait_send()

    return pl.pallas_call(
        body,
        out_shape=jax.ShapeDtypeStruct((B, S_PER, D), F32),
        in_specs=[pl.BlockSpec(memory_space=pltpu.VMEM)] * 5,
        out_specs=pl.BlockSpec(memory_space=pltpu.VMEM),
        scratch_shapes=[
            pltpu.VMEM((B, S, D), BF16),
            pltpu.VMEM((B, H_PER, S, DH), BF16),
            pltpu.VMEM((B, H_PER, S, DH), BF16),
            pltpu.VMEM((2, B, S_PER, D), F32),
            pltpu.VMEM((N_DEV - 1, B, S_PER, D), F32),
            pltpu.SemaphoreType.DMA((N_DEV - 1,)),
            pltpu.SemaphoreType.DMA((N_DEV - 1,)),
            pltpu.SemaphoreType.DMA((N_DEV - 1,)),
            pltpu.SemaphoreType.DMA((N_DEV - 1,)),
        ],
        compiler_params=pltpu.CompilerParams(
            collective_id=0, vmem_limit_bytes=100 * 1024 * 1024),
    )(x, wq_hm, wo_hm, wk_hm, wv_hm)


# device time: 287107 ns/iter; 1.5698x vs baseline; 1.2501x over previous
import jax
import jax.numpy as jnp
from jax import lax
from jax.experimental import pallas as pl
from jax.experimental.pallas import tpu as pltpu

N_DEV = 16
B = 2
S_PER = 128
S = S_PER * N_DEV
D = 512
H_PER = 8
DH = 64
SCALE = 0.125
BF16 = jnp.bfloat16
F32 = jnp.float32


def kernel(x, Wq, Wo, Wk, Wv):
    wq_hm = (Wq * SCALE).reshape(D, H_PER, DH).transpose(1, 0, 2).astype(BF16)
    wk_hm = Wk.reshape(D, H_PER, DH).transpose(1, 0, 2).astype(BF16)
    wv_hm = Wv.reshape(D, H_PER, DH).transpose(1, 0, 2).astype(BF16)
    wo_hm = Wo.reshape(H_PER, DH, D).astype(BF16)

    def body(x_ref, wq_ref, wo_ref, wk_ref, wv_ref, out_ref,
             xfull_ref, k_ref, v_ref, sendbuf_ref, rsrecv_ref,
             ag_send, ag_recv, rs_send, rs_recv):
        me = lax.axis_index("i")
        left = lax.rem(me - 1 + N_DEV, N_DEV)
        right = lax.rem(me + 1, N_DEV)

        barrier = pltpu.get_barrier_semaphore()
        for nbr in (left, right):
            pl.semaphore_signal(barrier, inc=1, device_id=(nbr,),
                                device_id_type=pl.DeviceIdType.MESH)
        pl.semaphore_wait(barrier, 2)

        def kv_chunk(c):
            for b in range(B):
                xc = xfull_ref[b, pl.ds(c * S_PER, S_PER), :]

                def kvb(hd, _):
                    k_ref[b, hd, pl.ds(c * S_PER, S_PER), :] = jnp.dot(
                        xc, wk_ref[hd],
                        preferred_element_type=F32).astype(BF16)
                    v_ref[b, hd, pl.ds(c * S_PER, S_PER), :] = jnp.dot(
                        xc, wv_ref[hd],
                        preferred_element_type=F32).astype(BF16)
                    return 0

                lax.fori_loop(0, H_PER, kvb, 0)

        xfull_ref[:, pl.ds(me * S_PER, S_PER), :] = x_ref[...].astype(BF16)
        ag_rdmas = []
        for h in range(N_DEV - 1):
            org = lax.rem(me - h + 2 * N_DEV, N_DEV)
            rdma = pltpu.make_async_remote_copy(
                src_ref=xfull_ref.at[:, pl.ds(org * S_PER, S_PER), :],
                dst_ref=xfull_ref.at[:, pl.ds(org * S_PER, S_PER), :],
                send_sem=ag_send.at[h],
                recv_sem=ag_recv.at[h],
                device_id=(right,),
                device_id_type=pl.DeviceIdType.MESH,
            )
            if h > 0:
                ag_rdmas[h - 1].wait_recv()
            rdma.start()
            ag_rdmas.append(rdma)
            kv_chunk(org)
        ag_rdmas[-1].wait_recv()
        kv_chunk(lax.rem(me + 1, N_DEV))

        rs_rdmas = []
        for t in range(N_DEV):
            c = lax.rem(me - t - 1 + 2 * N_DEV, N_DEV)
            contribs = []
            for b in range(B):
                xq = xfull_ref[b, pl.ds(c * S_PER, S_PER), :]

                def hd_body(hd, acc, b=b, xq=xq):
                    Qh = jnp.dot(xq, wq_ref[hd],
                                 preferred_element_type=F32).astype(BF16)
                    s = lax.dot_general(
                        Qh, k_ref[b, hd], (((1,), (1,)), ((), ())),
                        preferred_element_type=F32)
                    p = jnp.exp(s).astype(BF16)
                    l = jnp.sum(p, axis=1, keepdims=True, dtype=F32)
                    o = lax.dot_general(
                        p, v_ref[b, hd],
                        (((1,), (0,)), ((), ())),
                        preferred_element_type=F32) / l
                    return acc + jnp.dot(o.astype(BF16), wo_ref[hd],
                                         preferred_element_type=F32)

                contribs.append(lax.fori_loop(
                    0, H_PER, hd_body, jnp.zeros((S_PER, D), F32)))

            if t < N_DEV - 1:
                slot = t % 2
                if t >= 2:
                    rs_rdmas[t - 2].wait_send()
                if t > 0:
                    rs_rdmas[t - 1].wait_recv()
                for b in range(B):
                    acc = contribs[b]
                    if t > 0:
                        acc = acc + rsrecv_ref[t - 1, b].astype(F32)
                    sendbuf_ref[slot, b] = acc.astype(BF16)
                rdma = pltpu.make_async_remote_copy(
                    src_ref=sendbuf_ref.at[slot],
                    dst_ref=rsrecv_ref.at[t],
                    send_sem=rs_send.at[t],
                    recv_sem=rs_recv.at[t],
                    device_id=(right,),
                    device_id_type=pl.DeviceIdType.MESH,
                )
                rdma.start()
                rs_rdmas.append(rdma)
            else:
                rs_rdmas[N_DEV - 2].wait_recv()
                for b in range(B):
                    out_ref[b] = (contribs[b]
                                  + rsrecv_ref[N_DEV - 2, b].astype(F32))

        for r in ag_rdmas:
            r.wait_send()
        rs_rdmas[-2].wait_send()
        rs_rdmas[-1].wait_send()

    return pl.pallas_call(
        body,
        out_shape=jax.ShapeDtypeStruct((B, S_PER, D), F32),
        in_specs=[pl.BlockSpec(memory_space=pltpu.VMEM)] * 5,
        out_specs=pl.BlockSpec(memory_space=pltpu.VMEM),
        scratch_shapes=[
            pltpu.VMEM((B, S, D), BF16),
            pltpu.VMEM((B, H_PER, S, DH), BF16),
            pltpu.VMEM((B, H_PER, S, DH), BF16),
            pltpu.VMEM((2, B, S_PER, D), BF16),
            pltpu.VMEM((N_DEV - 1, B, S_PER, D), BF16),
            pltpu.SemaphoreType.DMA((N_DEV - 1,)),
            pltpu.SemaphoreType.DMA((N_DEV - 1,)),
            pltpu.SemaphoreType.DMA((N_DEV - 1,)),
            pltpu.SemaphoreType.DMA((N_DEV - 1,)),
        ],
        compiler_params=pltpu.CompilerParams(
            collective_id=0, vmem_limit_bytes=100 * 1024 * 1024),
    )(x, wq_hm, wo_hm, wk_hm, wv_hm)


# device time: 251877 ns/iter; 1.7893x vs baseline; 1.1399x over previous
import jax
import jax.numpy as jnp
from jax import lax
from jax.experimental import pallas as pl
from jax.experimental.pallas import tpu as pltpu

N_DEV = 16
B = 2
S_PER = 128
S = S_PER * N_DEV
D = 512
H_PER = 8
DH = 64
SCALE = 0.125
BF16 = jnp.bfloat16
F32 = jnp.float32


def kernel(x, Wq, Wo, Wk, Wv):
    wq_hm = (Wq * SCALE).reshape(D, H_PER, DH).transpose(1, 0, 2).astype(BF16)
    wk16 = Wk.astype(BF16)
    wv16 = Wv.astype(BF16)
    wo_hm = Wo.reshape(H_PER, DH, D).astype(BF16)

    def body(x_ref, wq_ref, wo_ref, wk_ref, wv_ref, out_ref,
             xfull_ref, k_ref, v_ref, sendbuf_ref, rsrecv_ref,
             ag_send_r, ag_recv_r, ag_send_l, ag_recv_l,
             rs_send, rs_recv):
        me = lax.axis_index("i")
        left = lax.rem(me - 1 + N_DEV, N_DEV)
        right = lax.rem(me + 1, N_DEV)

        barrier = pltpu.get_barrier_semaphore()
        for nbr in (left, right):
            pl.semaphore_signal(barrier, inc=1, device_id=(nbr,),
                                device_id_type=pl.DeviceIdType.MESH)
        pl.semaphore_wait(barrier, 2)

        def kv_chunk(c):
            for b in range(B):
                xc = xfull_ref[b, pl.ds(c * S_PER, S_PER), :]
                kf = jnp.dot(xc, wk_ref[...],
                             preferred_element_type=F32).astype(BF16)
                vf = jnp.dot(xc, wv_ref[...],
                             preferred_element_type=F32).astype(BF16)
                for hd in range(H_PER):
                    k_ref[b, hd, pl.ds(c * S_PER, S_PER), :] = (
                        kf[:, hd * DH:(hd + 1) * DH])
                    v_ref[b, hd, pl.ds(c * S_PER, S_PER), :] = (
                        vf[:, hd * DH:(hd + 1) * DH])

        xfull_ref[:, pl.ds(me * S_PER, S_PER), :] = x_ref[...].astype(BF16)
        r_rdmas, l_rdmas = [], []
        NR, NL = 8, 7
        for h in range(NR):
            org_r = lax.rem(me - h + 2 * N_DEV, N_DEV)
            rd = pltpu.make_async_remote_copy(
                src_ref=xfull_ref.at[:, pl.ds(org_r * S_PER, S_PER), :],
                dst_ref=xfull_ref.at[:, pl.ds(org_r * S_PER, S_PER), :],
                send_sem=ag_send_r.at[h],
                recv_sem=ag_recv_r.at[h],
                device_id=(right,),
                device_id_type=pl.DeviceIdType.MESH,
            )
            if h > 0:
                r_rdmas[h - 1].wait_recv()
            rd.start()
            r_rdmas.append(rd)
            if h < NL:
                org_l = lax.rem(me + h, N_DEV)
                ld = pltpu.make_async_remote_copy(
                    src_ref=xfull_ref.at[:, pl.ds(org_l * S_PER, S_PER), :],
                    dst_ref=xfull_ref.at[:, pl.ds(org_l * S_PER, S_PER), :],
                    send_sem=ag_send_l.at[h],
                    recv_sem=ag_recv_l.at[h],
                    device_id=(left,),
                    device_id_type=pl.DeviceIdType.MESH,
                )
                if h > 0:
                    l_rdmas[h - 1].wait_recv()
                ld.start()
                l_rdmas.append(ld)
            elif h > 0 and h - 1 < NL:
                l_rdmas[h - 1].wait_recv()
            if h == 0:
                kv_chunk(me)
            else:
                kv_chunk(lax.rem(me - h + 2 * N_DEV, N_DEV))
                kv_chunk(lax.rem(me + h, N_DEV))
        r_rdmas[NR - 1].wait_recv()
        kv_chunk(lax.rem(me - NR + 2 * N_DEV, N_DEV))

        rs_rdmas = []
        for t in range(N_DEV):
            c = lax.rem(me - t - 1 + 2 * N_DEV, N_DEV)
            contribs = []
            for b in range(B):
                xq = xfull_ref[b, pl.ds(c * S_PER, S_PER), :]

                def hd_body(hd, acc, b=b, xq=xq):
                    Qh = jnp.dot(xq, wq_ref[hd],
                                 preferred_element_type=F32).astype(BF16)
                    s = lax.dot_general(
                        Qh, k_ref[b, hd], (((1,), (1,)), ((), ())),
                        preferred_element_type=F32)
                    p = jnp.exp(s).astype(BF16)
                    l = jnp.sum(p, axis=1, keepdims=True, dtype=F32)
                    o = lax.dot_general(
                        p, v_ref[b, hd],
                        (((1,), (0,)), ((), ())),
                        preferred_element_type=F32) / l
                    return acc + jnp.dot(o.astype(BF16), wo_ref[hd],
                                         preferred_element_type=F32)

                contribs.append(lax.fori_loop(
                    0, H_PER, hd_body, jnp.zeros((S_PER, D), F32)))

            if t < N_DEV - 1:
                slot = t % 2
                if t >= 2:
                    rs_rdmas[t - 2].wait_send()
                if t > 0:
                    rs_rdmas[t - 1].wait_recv()
                for b in range(B):
                    acc = contribs[b]
                    if t > 0:
                        acc = acc + rsrecv_ref[t - 1, b].astype(F32)
                    sendbuf_ref[slot, b] = acc.astype(BF16)
                rdma = pltpu.make_async_remote_copy(
                    src_ref=sendbuf_ref.at[slot],
                    dst_ref=rsrecv_ref.at[t],
                    send_sem=rs_send.at[t],
                    recv_sem=rs_recv.at[t],
                    device_id=(right,),
                    device_id_type=pl.DeviceIdType.MESH,
                )
                rdma.start()
                rs_rdmas.append(rdma)
            else:
                rs_rdmas[N_DEV - 2].wait_recv()
                for b in range(B):
                    out_ref[b] = (contribs[b]
                                  + rsrecv_ref[N_DEV - 2, b].astype(F32))

        for r in r_rdmas + l_rdmas:
            r.wait_send()
        rs_rdmas[-2].wait_send()
        rs_rdmas[-1].wait_send()

    return pl.pallas_call(
        body,
        out_shape=jax.ShapeDtypeStruct((B, S_PER, D), F32),
        in_specs=[pl.BlockSpec(memory_space=pltpu.VMEM)] * 5,
        out_specs=pl.BlockSpec(memory_space=pltpu.VMEM),
        scratch_shapes=[
            pltpu.VMEM((B, S, D), BF16),
            pltpu.VMEM((B, H_PER, S, DH), BF16),
            pltpu.VMEM((B, H_PER, S, DH), BF16),
            pltpu.VMEM((2, B, S_PER, D), BF16),
            pltpu.VMEM((N_DEV - 1, B, S_PER, D), BF16),
            pltpu.SemaphoreType.DMA((8,)),
            pltpu.SemaphoreType.DMA((8,)),
            pltpu.SemaphoreType.DMA((7,)),
            pltpu.SemaphoreType.DMA((7,)),
            pltpu.SemaphoreType.DMA((N_DEV - 1,)),
            pltpu.SemaphoreType.DMA((N_DEV - 1,)),
        ],
        compiler_params=pltpu.CompilerParams(
            collective_id=0, vmem_limit_bytes=100 * 1024 * 1024),
    )(x, wq_hm, wo_hm, wk16, wv16)


# device time: 236307 ns/iter; 1.9072x vs baseline; 1.0659x over previous
import jax
import jax.numpy as jnp
from jax import lax
from jax.experimental import pallas as pl
from jax.experimental.pallas import tpu as pltpu

N_DEV = 16
B = 2
S_PER = 128
S = S_PER * N_DEV
D = 512
H_PER = 8
DH = 64
SCALE = 0.125
BF16 = jnp.bfloat16
F32 = jnp.float32


def kernel(x, Wq, Wo, Wk, Wv):
    wq_hm = (Wq * SCALE).reshape(D, H_PER, DH).transpose(1, 0, 2).astype(BF16)
    wk16 = Wk.astype(BF16)
    wv16 = Wv.astype(BF16)
    wo_hm = Wo.reshape(H_PER, DH, D).astype(BF16)

    def body(x_ref, wq_ref, wo_ref, wk_ref, wv_ref, out_ref,
             xfull_ref, k_ref, v_ref, sendbuf_ref, rsrecv_ref,
             ag_send_r, ag_recv_r, ag_send_l, ag_recv_l,
             rs_send, rs_recv):
        me = lax.axis_index("i")
        left = lax.rem(me - 1 + N_DEV, N_DEV)
        right = lax.rem(me + 1, N_DEV)

        barrier = pltpu.get_barrier_semaphore()
        for nbr in (left, right):
            pl.semaphore_signal(barrier, inc=1, device_id=(nbr,),
                                device_id_type=pl.DeviceIdType.MESH)
        pl.semaphore_wait(barrier, 2)

        def kv_chunk(c):
            for b in range(B):
                xc = xfull_ref[b, pl.ds(c * S_PER, S_PER), :]
                kf = jnp.dot(xc, wk_ref[...],
                             preferred_element_type=F32).astype(BF16)
                vf = jnp.dot(xc, wv_ref[...],
                             preferred_element_type=F32).astype(BF16)
                for hd in range(H_PER):
                    k_ref[b, hd, pl.ds(c * S_PER, S_PER), :] = (
                        kf[:, hd * DH:(hd + 1) * DH])
                    v_ref[b, hd, pl.ds(c * S_PER, S_PER), :] = (
                        vf[:, hd * DH:(hd + 1) * DH])

        xfull_ref[:, pl.ds(me * S_PER, S_PER), :] = x_ref[...].astype(BF16)
        r_rdmas, l_rdmas = [], []
        NR, NL = 8, 7
        for h in range(NR):
            org_r = lax.rem(me - h + 2 * N_DEV, N_DEV)
            rd = pltpu.make_async_remote_copy(
                src_ref=xfull_ref.at[:, pl.ds(org_r * S_PER, S_PER), :],
                dst_ref=xfull_ref.at[:, pl.ds(org_r * S_PER, S_PER), :],
                send_sem=ag_send_r.at[h],
                recv_sem=ag_recv_r.at[h],
                device_id=(right,),
                device_id_type=pl.DeviceIdType.MESH,
            )
            if h > 0:
                r_rdmas[h - 1].wait_recv()
            rd.start()
            r_rdmas.append(rd)
            if h < NL:
                org_l = lax.rem(me + h, N_DEV)
                ld = pltpu.make_async_remote_copy(
                    src_ref=xfull_ref.at[:, pl.ds(org_l * S_PER, S_PER), :],
                    dst_ref=xfull_ref.at[:, pl.ds(org_l * S_PER, S_PER), :],
                    send_sem=ag_send_l.at[h],
                    recv_sem=ag_recv_l.at[h],
                    device_id=(left,),
                    device_id_type=pl.DeviceIdType.MESH,
                )
                if h > 0:
                    l_rdmas[h - 1].wait_recv()
                ld.start()
                l_rdmas.append(ld)
            elif h > 0 and h - 1 < NL:
                l_rdmas[h - 1].wait_recv()
            if h == 0:
                kv_chunk(me)
            else:
                kv_chunk(lax.rem(me - h + 2 * N_DEV, N_DEV))
                kv_chunk(lax.rem(me + h, N_DEV))
        r_rdmas[NR - 1].wait_recv()
        kv_chunk(lax.rem(me - NR + 2 * N_DEV, N_DEV))

        rs_rdmas = []
        for t in range(N_DEV):
            c = lax.rem(me - t - 1 + 2 * N_DEV, N_DEV)
            xqs = [xfull_ref[b, pl.ds(c * S_PER, S_PER), :] for b in range(B)]

            def hd_body(hd, carry, xqs=xqs):
                outs = []
                for b in range(B):
                    Qh = jnp.dot(xqs[b], wq_ref[hd],
                                 preferred_element_type=F32).astype(BF16)
                    s = lax.dot_general(
                        Qh, k_ref[b, hd], (((1,), (1,)), ((), ())),
                        preferred_element_type=F32)
                    p = jnp.exp(s).astype(BF16)
                    l = jnp.sum(p, axis=1, keepdims=True, dtype=F32)
                    o = lax.dot_general(
                        p, v_ref[b, hd],
                        (((1,), (0,)), ((), ())),
                        preferred_element_type=F32) / l
                    outs.append(carry[b] + jnp.dot(
                        o.astype(BF16), wo_ref[hd],
                        preferred_element_type=F32))
                return tuple(outs)

            contribs = list(lax.fori_loop(
                0, H_PER, hd_body,
                tuple(jnp.zeros((S_PER, D), F32) for _ in range(B))))

            if t < N_DEV - 1:
                slot = t % 2
                if t >= 2:
                    rs_rdmas[t - 2].wait_send()
                if t > 0:
                    rs_rdmas[t - 1].wait_recv()
                for b in range(B):
                    acc = contribs[b]
                    if t > 0:
                        acc = acc + rsrecv_ref[t - 1, b].astype(F32)
                    sendbuf_ref[slot, b] = acc.astype(BF16)
                rdma = pltpu.make_async_remote_copy(
                    src_ref=sendbuf_ref.at[slot],
                    dst_ref=rsrecv_ref.at[t],
                    send_sem=rs_send.at[t],
                    recv_sem=rs_recv.at[t],
                    device_id=(right,),
                    device_id_type=pl.DeviceIdType.MESH,
                )
                rdma.start()
                rs_rdmas.append(rdma)
            else:
                rs_rdmas[N_DEV - 2].wait_recv()
                for b in range(B):
                    out_ref[b] = (contribs[b]
                                  + rsrecv_ref[N_DEV - 2, b].astype(F32))

        for r in r_rdmas + l_rdmas:
            r.wait_send()
        rs_rdmas[-2].wait_send()
        rs_rdmas[-1].wait_send()

    return pl.pallas_call(
        body,
        out_shape=jax.ShapeDtypeStruct((B, S_PER, D), F32),
        in_specs=[pl.BlockSpec(memory_space=pltpu.VMEM)] * 5,
        out_specs=pl.BlockSpec(memory_space=pltpu.VMEM),
        scratch_shapes=[
            pltpu.VMEM((B, S, D), BF16),
            pltpu.VMEM((B, H_PER, S, DH), BF16),
            pltpu.VMEM((B, H_PER, S, DH), BF16),
            pltpu.VMEM((2, B, S_PER, D), BF16),
            pltpu.VMEM((N_DEV - 1, B, S_PER, D), BF16),
            pltpu.SemaphoreType.DMA((8,)),
            pltpu.SemaphoreType.DMA((8,)),
            pltpu.SemaphoreType.DMA((7,)),
            pltpu.SemaphoreType.DMA((7,)),
            pltpu.SemaphoreType.DMA((N_DEV - 1,)),
            pltpu.SemaphoreType.DMA((N_DEV - 1,)),
        ],
        compiler_params=pltpu.CompilerParams(
            collective_id=0, vmem_limit_bytes=100 * 1024 * 1024),
    )(x, wq_hm, wo_hm, wk16, wv16)
